# initial kernel scaffold (unmeasured)
import jax
import jax.numpy as jnp
from jax import lax
from jax.experimental import pallas as pl
from jax.experimental.pallas import tpu as pltpu

N_DEV = 8


def kernel(x, w_mat):
    k_glob, k_blk = x.shape
    _, n = w_mat.shape
    m_blk = k_glob // N_DEV

    def body(x_ref, w_ref, out_ref, xb_ref, comm_ref, send_sems, recv_sems):
        my = lax.axis_index("i")

        xb_ref[:, :] = x_ref[:, :].astype(jnp.bfloat16)

        rdmas = []
        for k in range(1, N_DEV):
            dst = (my + k) % N_DEV
            rdma = pltpu.make_async_remote_copy(
                src_ref=xb_ref.at[pl.ds(dst * m_blk, m_blk), :],
                dst_ref=comm_ref.at[k - 1],
                send_sem=send_sems.at[k - 1],
                recv_sem=recv_sems.at[k - 1],
                device_id=(dst,),
                device_id_type=pl.DeviceIdType.MESH,
            )
            rdma.start()
            rdmas.append(rdma)

        def w_slice(idx):
            return w_ref[pl.ds(idx * m_blk, m_blk), :].astype(jnp.bfloat16)

        acc = jnp.dot(
            xb_ref[pl.ds(my * m_blk, m_blk), :],
            w_slice(my),
            preferred_element_type=jnp.float32,
        )
        for k in range(1, N_DEV):
            rdmas[k - 1].wait_recv()
            src = (my - k) % N_DEV
            acc = acc + jnp.dot(
                comm_ref[k - 1],
                w_slice(src),
                preferred_element_type=jnp.float32,
            )

        c = 0.7978845608028654
        out_ref[:, :] = 0.5 * acc * (
            1.0 + jnp.tanh(c * (acc + 0.044715 * acc * acc * acc))
        )

        for k in range(1, N_DEV):
            rdmas[k - 1].wait_send()

    return pl.pallas_call(
        body,
        out_shape=jax.ShapeDtypeStruct((m_blk, n), jnp.float32),
        in_specs=[
            pl.BlockSpec(memory_space=pltpu.VMEM),
            pl.BlockSpec(memory_space=pltpu.VMEM),
        ],
        out_specs=pl.BlockSpec(memory_space=pltpu.VMEM),
        scratch_shapes=[
            pltpu.VMEM((k_glob, k_blk), jnp.bfloat16),
            pltpu.VMEM((N_DEV - 1, m_blk, k_blk), jnp.bfloat16),
            pltpu.SemaphoreType.DMA((N_DEV - 1,)),
            pltpu.SemaphoreType.DMA((N_DEV - 1,)),
        ],
        compiler_params=pltpu.CompilerParams(collective_id=0),
    )(x, w_mat)


# baseline (device time: 25630 ns/iter reference)
import jax
import jax.numpy as jnp
from jax import lax
from jax.experimental import pallas as pl
from jax.experimental.pallas import tpu as pltpu

N_DEV = 8


def kernel(x, w_mat):
    k_glob, k_blk = x.shape
    _, n = w_mat.shape
    m_blk = k_glob // N_DEV

    def body(x_ref, w_ref, out_ref, xb_ref, comm_ref, send_sems, recv_sems):
        my = lax.axis_index("i")

        xb_ref[:, :] = x_ref[:, :].astype(jnp.bfloat16)

        rdmas = []
        for k in range(1, N_DEV):
            dst = (my + k) % N_DEV
            rdma = pltpu.make_async_remote_copy(
                src_ref=xb_ref.at[pl.ds(dst * m_blk, m_blk), :],
                dst_ref=comm_ref.at[k - 1],
                send_sem=send_sems.at[k - 1],
                recv_sem=recv_sems.at[k - 1],
                device_id=(dst,),
                device_id_type=pl.DeviceIdType.MESH,
            )
            rdma.start()
            rdmas.append(rdma)

        def w_slice(idx):
            return w_ref[pl.ds(idx * m_blk, m_blk), :].astype(jnp.bfloat16)

        acc = jnp.dot(
            xb_ref[pl.ds(my * m_blk, m_blk), :],
            w_slice(my),
            preferred_element_type=jnp.float32,
        )
        for k in range(1, N_DEV):
            rdmas[k - 1].wait_recv()
            src = (my - k) % N_DEV
            acc = acc + jnp.dot(
                comm_ref[k - 1],
                w_slice(src),
                preferred_element_type=jnp.float32,
            )

        c = 0.7978845608028654
        out_ref[:, :] = 0.5 * acc * (
            1.0 + jnp.tanh(c * (acc + 0.044715 * acc * acc * acc))
        )

        for k in range(1, N_DEV):
            rdmas[k - 1].wait_send()

    return pl.pallas_call(
        body,
        out_shape=jax.ShapeDtypeStruct((m_blk, n), jnp.float32),
        in_specs=[
            pl.BlockSpec(memory_space=pltpu.VMEM),
            pl.BlockSpec(memory_space=pltpu.VMEM),
        ],
        out_specs=pl.BlockSpec(memory_space=pltpu.VMEM),
        scratch_shapes=[
            pltpu.VMEM((k_glob, k_blk), jnp.bfloat16),
            pltpu.VMEM((N_DEV - 1, m_blk, k_blk), jnp.bfloat16),
            pltpu.SemaphoreType.DMA((N_DEV - 1,)),
            pltpu.SemaphoreType.DMA((N_DEV - 1,)),
        ],
    )(x, w_mat)


# device time: 11981 ns/iter; 2.1392x vs baseline; 2.1392x over previous
import jax
import jax.numpy as jnp
from jax import lax
from jax.experimental import pallas as pl
from jax.experimental.pallas import tpu as pltpu

N_DEV = 8


def kernel(x, w_mat):
    k_glob, k_blk = x.shape
    _, n = w_mat.shape
    m_blk = k_glob // N_DEV

    def body(x_ref, w_ref, out_ref, xb_ref, comm_ref, send_sems, recv_sems):
        my = lax.axis_index("i")

        xb_ref[:, :] = x_ref[:, :].astype(jnp.bfloat16)

        ABLATE_COMM = True
        rdmas = []
        if not ABLATE_COMM:
            for k in range(1, N_DEV):
                dst = (my + k) % N_DEV
                rdma = pltpu.make_async_remote_copy(
                    src_ref=xb_ref.at[pl.ds(dst * m_blk, m_blk), :],
                    dst_ref=comm_ref.at[k - 1],
                    send_sem=send_sems.at[k - 1],
                    recv_sem=recv_sems.at[k - 1],
                    device_id=(dst,),
                    device_id_type=pl.DeviceIdType.MESH,
                )
                rdma.start()
                rdmas.append(rdma)

        def w_slice(idx):
            return w_ref[pl.ds(idx * m_blk, m_blk), :].astype(jnp.bfloat16)

        acc = jnp.dot(
            xb_ref[pl.ds(my * m_blk, m_blk), :],
            w_slice(my),
            preferred_element_type=jnp.float32,
        )
        for k in range(1, N_DEV):
            src = (my - k) % N_DEV
            if not ABLATE_COMM:
                rdmas[k - 1].wait_recv()
                blk = comm_ref[k - 1]
            else:
                blk = xb_ref[pl.ds(src * m_blk, m_blk), :]
            acc = acc + jnp.dot(
                blk,
                w_slice(src),
                preferred_element_type=jnp.float32,
            )

        c = 0.7978845608028654
        out_ref[:, :] = 0.5 * acc * (
            1.0 + jnp.tanh(c * (acc + 0.044715 * acc * acc * acc))
        )

        if not ABLATE_COMM:
            for k in range(1, N_DEV):
                rdmas[k - 1].wait_send()

    return pl.pallas_call(
        body,
        out_shape=jax.ShapeDtypeStruct((m_blk, n), jnp.float32),
        in_specs=[
            pl.BlockSpec(memory_space=pltpu.VMEM),
            pl.BlockSpec(memory_space=pltpu.VMEM),
        ],
        out_specs=pl.BlockSpec(memory_space=pltpu.VMEM),
        scratch_shapes=[
            pltpu.VMEM((k_glob, k_blk), jnp.bfloat16),
            pltpu.VMEM((N_DEV - 1, m_blk, k_blk), jnp.bfloat16),
            pltpu.SemaphoreType.DMA((N_DEV - 1,)),
            pltpu.SemaphoreType.DMA((N_DEV - 1,)),
        ],
    )(x, w_mat)


# device time: 11518 ns/iter; 2.2252x vs baseline; 1.0402x over previous
import os

import jax
import jax.numpy as jnp
from jax import lax
from jax.experimental import pallas as pl
from jax.experimental.pallas import tpu as pltpu

N_DEV = 8

ABLATE_COMM = os.environ.get("ABL_COMM") == "1"
ABLATE_GELU = os.environ.get("ABL_GELU") == "1"
ABLATE_WCAST = os.environ.get("ABL_WCAST") == "1"


def kernel(x, w_mat):
    k_glob, k_blk = x.shape
    _, n = w_mat.shape
    m_blk = k_glob // N_DEV

    def body(x_ref, w_ref, out_ref, xb_ref, comm_ref, send_sems, recv_sems):
        my = lax.axis_index("i")

        xb_ref[:, :] = x_ref[:, :].astype(jnp.bfloat16)

        rdmas = []
        if not ABLATE_COMM:
            for k in range(1, N_DEV):
                dst = (my + k) % N_DEV
                rdma = pltpu.make_async_remote_copy(
                    src_ref=xb_ref.at[pl.ds(dst * m_blk, m_blk), :],
                    dst_ref=comm_ref.at[k - 1],
                    send_sem=send_sems.at[k - 1],
                    recv_sem=recv_sems.at[k - 1],
                    device_id=(dst,),
                    device_id_type=pl.DeviceIdType.MESH,
                )
                rdma.start()
                rdmas.append(rdma)

        def w_slice(idx):
            return w_ref[pl.ds(idx * m_blk, m_blk), :].astype(jnp.bfloat16)

        w0 = w_slice(0) if ABLATE_WCAST else None

        acc = jnp.dot(
            xb_ref[pl.ds(my * m_blk, m_blk), :],
            w0 if ABLATE_WCAST else w_slice(my),
            preferred_element_type=jnp.float32,
        )
        for k in range(1, N_DEV):
            src = (my - k) % N_DEV
            if not ABLATE_COMM:
                rdmas[k - 1].wait_recv()
                blk = comm_ref[k - 1]
            else:
                blk = xb_ref[pl.ds(src * m_blk, m_blk), :]
            acc = acc + jnp.dot(
                blk,
                w0 if ABLATE_WCAST else w_slice(src),
                preferred_element_type=jnp.float32,
            )

        if ABLATE_GELU:
            out_ref[:, :] = acc
        else:
            c = 0.7978845608028654
            out_ref[:, :] = 0.5 * acc * (
                1.0 + jnp.tanh(c * (acc + 0.044715 * acc * acc * acc))
            )

        if not ABLATE_COMM:
            for k in range(1, N_DEV):
                rdmas[k - 1].wait_send()

    return pl.pallas_call(
        body,
        out_shape=jax.ShapeDtypeStruct((m_blk, n), jnp.float32),
        in_specs=[
            pl.BlockSpec(memory_space=pltpu.VMEM),
            pl.BlockSpec(memory_space=pltpu.VMEM),
        ],
        out_specs=pl.BlockSpec(memory_space=pltpu.VMEM),
        scratch_shapes=[
            pltpu.VMEM((k_glob, k_blk), jnp.bfloat16),
            pltpu.VMEM((N_DEV - 1, m_blk, k_blk), jnp.bfloat16),
            pltpu.SemaphoreType.DMA((N_DEV - 1,)),
            pltpu.SemaphoreType.DMA((N_DEV - 1,)),
        ],
    )(x, w_mat)
